# baseline (speedup 1.0000x reference)
import functools

import jax
import jax.numpy as jnp
from jax import lax
from jax.experimental import pallas as pl
from jax.experimental.pallas import tpu as pltpu
from jax.experimental.pallas import tpu_sc as plsc

N_NODES = 10000
N_EDGES = 320000
NP = 10112
STRIPE = NP // 16
CHUNK = 128
ROWS = N_EDGES // CHUNK
NW = 32
ROWS_PAD = 2560
RPW = ROWS_PAD // NW
EP = ROWS_PAD * CHUNK
L = 16

f32 = jnp.float32


EROWS = N_EDGES // 8


def _edge_proj1(ea_r, wbd1, bb1):
    BE = 2000
    blk = pl.BlockSpec((BE, 128), lambda i: (i, 0))
    full = lambda shp: pl.BlockSpec(shp, lambda i: (0, 0))

    def body(ea_ref, w1_ref, b1_ref, o_ref):
        o_ref[...] = (jnp.dot(ea_ref[...], w1_ref[...],
                              preferred_element_type=f32) + b1_ref[...])

    return pl.pallas_call(
        body,
        grid=(EROWS // BE,),
        in_specs=[blk, full((128, 128)), full((1, 128))],
        out_specs=blk,
        out_shape=jax.ShapeDtypeStruct((EROWS, 128), f32),
    )(ea_r, wbd1, bb1)


def _edge_proj23(ea_r, wbd2, bb2, wbd3, bb3):
    BE = 2000
    blk = pl.BlockSpec((BE, 128), lambda i: (i, 0))
    full = lambda shp: pl.BlockSpec(shp, lambda i: (0, 0))

    def body(ea_ref, w2_ref, b2_ref, w3_ref, b3_ref, *outs):
        z = ea_ref[...]
        for x in range(2):
            outs[x][...] = jnp.dot(z[:, 64 * x:64 * x + 64], w2_ref[...],
                                   preferred_element_type=f32) + b2_ref[...]
        for x in range(4):
            outs[2 + x][...] = jnp.dot(z[:, 32 * x:32 * x + 32], w3_ref[...],
                                       preferred_element_type=f32) + b3_ref[...]

    return pl.pallas_call(
        body,
        grid=(EROWS // BE,),
        in_specs=[blk, full((64, 128)), full((1, 128)),
                  full((32, 128)), full((1, 128))],
        out_specs=[blk] * 6,
        out_shape=[jax.ShapeDtypeStruct((EROWS, 128), f32)] * 6,
    )(ea_r, wbd2, bb2, wbd3, bb3)


def _node_proj_first(x, WiT, WjT, UT, cb):
    d = WiT.shape[1]

    def body(x_ref, wi_ref, wj_ref, u_ref, c_ref, a_ref, b_ref, s_ref):
        xb = x_ref[...]
        a_ref[...] = jnp.dot(xb, wi_ref[...], preferred_element_type=f32)
        b_ref[...] = jnp.dot(xb, wj_ref[...], preferred_element_type=f32)
        s_ref[...] = jnp.dot(xb, u_ref[...], preferred_element_type=f32) + c_ref[...]

    return pl.pallas_call(
        body,
        out_shape=[jax.ShapeDtypeStruct((NP, d), f32)] * 3,
    )(x, WiT, WjT, UT, cb)


def _node_proj_next(acc, s_prev, WiT, WjT, UT, cb):
    d = WiT.shape[1]

    def body(acc_ref, sp_ref, wi_ref, wj_ref, u_ref, c_ref, a_ref, b_ref, s_ref):
        xb = acc_ref[0] + acc_ref[1] + sp_ref[...]
        a_ref[...] = jnp.dot(xb, wi_ref[...], preferred_element_type=f32)
        b_ref[...] = jnp.dot(xb, wj_ref[...], preferred_element_type=f32)
        s_ref[...] = jnp.dot(xb, u_ref[...], preferred_element_type=f32) + c_ref[...]

    return pl.pallas_call(
        body,
        out_shape=[jax.ShapeDtypeStruct((NP, d), f32)] * 3,
    )(acc, s_prev, WiT, WjT, UT, cb)


def _combine(acc, s3):
    def body(acc_ref, s_ref, o_ref):
        t = acc_ref[0] + acc_ref[1] + s_ref[...]
        o_ref[...] = t[:N_NODES]

    return pl.pallas_call(
        body,
        out_shape=jax.ShapeDtypeStruct((N_NODES, 64), f32),
    )(acc, s3)


def _sc_message_pass(A, B, Es, dstr, srcr, d):
    KV = d // L
    P = d // 16
    EPR = 8 // P
    mesh = plsc.VectorSubcoreMesh(
        core_axis_name="c", subcore_axis_name="s", num_cores=2, num_subcores=16)

    NBG = 3 if d == 64 else 4
    QR = 10

    def body(*refs):
        (a_hbm, b_hbm), e_hbms = refs[:2], refs[2:2 + P]
        dst_hbm, src_hbm, out_hbm, acc, abuf, bbuf, mbuf = refs[2 + P:9 + P]
        ebufs = refs[9 + P:9 + 2 * P]
        idxd, idxs, sa, sb, se, ss = refs[9 + 2 * P:]
        cid = lax.axis_index("c")
        sid = lax.axis_index("s")
        wid = cid * 16 + sid
        base_r = wid * RPW

        def zrow(i, carry):
            for k in range(KV):
                mbuf[0, i, pl.ds(k * L, L)] = jnp.zeros((L,), f32)
            return carry
        lax.fori_loop(0, CHUNK, zrow, 0)
        base = sid * STRIPE
        for t in range(STRIPE // CHUNK):
            pltpu.sync_copy(mbuf.at[0], acc.at[pl.ds(base + t * CHUNK, CHUNK)])
        rem = STRIPE % CHUNK
        pltpu.sync_copy(mbuf.at[0, pl.ds(0, rem)],
                        acc.at[pl.ds(base + STRIPE - rem, rem)])

        def load_octet(q):
            qs = q % 2
            pltpu.sync_copy(dst_hbm.at[pl.ds(base_r + q * QR, QR)], idxd.at[qs])
            pltpu.sync_copy(src_hbm.at[pl.ds(base_r + q * QR, QR)], idxs.at[qs])

        def drow(g):
            return idxd.at[(g // QR) % 2, g % QR]

        def srow(g):
            return idxs.at[(g // QR) % 2, g % QR]

        def issue_ab(g):
            gb = g % NBG
            pltpu.async_copy(a_hbm.at[drow(g)], abuf.at[gb], sa.at[gb])
            pltpu.async_copy(b_hbm.at[srow(g)], bbuf.at[gb], sb.at[gb])

        def issue_e(g):
            eb = g % 2

            @pl.when(base_r + g < ROWS)
            def _():
                for x in range(P):
                    pltpu.async_copy(
                        e_hbms[x].at[pl.ds((base_r + g) * 16, 16)],
                        ebufs[x].at[eb], se.at[eb])

        load_octet(0)
        plsc.subcore_barrier()
        for g0 in range(NBG):
            issue_ab(g0)
        for g0 in range(2):
            issue_e(g0)

        def step(g, carry):
            gb = g % NBG
            mb = g % 2
            pltpu.make_async_copy(a_hbm.at[drow(g)], abuf.at[gb],
                                  sa.at[gb]).wait()
            pltpu.make_async_copy(b_hbm.at[srow(g)], bbuf.at[gb],
                                  sb.at[gb]).wait()

            @pl.when(base_r + g < ROWS)
            def _():
                for x in range(P):
                    pltpu.make_async_copy(
                        e_hbms[x].at[pl.ds((base_r + g) * 16, 16)],
                        ebufs[x].at[mb], se.at[mb]).wait()

            @pl.when(g >= 2)
            def _():
                pltpu.make_async_copy(
                    mbuf.at[mb], acc.at[drow(lax.max(g - 2, 0))],
                    ss.at[mb]).wait()

            def crow(t, c2):
                for x in range(P):
                    for u in range(EPR):
                        e_loc = t * 8 + x * EPR + u
                        for k in range(KV):
                            sl = pl.ds(k * L, L)
                            v = (abuf[gb, e_loc, sl] + bbuf[gb, e_loc, sl]
                                 + ebufs[x][mb, t, pl.ds(u * d + k * L, L)])
                            mbuf[mb, e_loc, sl] = jnp.where(v >= 0.0, v, v * 0.01)
                return c2
            lax.fori_loop(0, 16, crow, 0)

            pltpu.async_copy(mbuf.at[mb], acc.at[drow(g)], ss.at[mb], add=True)

            @pl.when(((g + NBG) % QR == 0) & (g + NBG < RPW))
            def _():
                load_octet((g + NBG) // QR)

            @pl.when(g + NBG < RPW)
            def _():
                issue_ab(g + NBG)

            @pl.when(g + 2 < RPW)
            def _():
                issue_e(g + 2)
            return carry
        lax.fori_loop(0, RPW, step, 0)

        for g in (RPW - 2, RPW - 1):
            pltpu.make_async_copy(mbuf.at[g % 2], acc.at[drow(g)],
                                  ss.at[g % 2]).wait()

        plsc.subcore_barrier()
        pltpu.sync_copy(acc.at[pl.ds(sid * STRIPE, STRIPE)],
                        out_hbm.at[cid, pl.ds(sid * STRIPE, STRIPE)])

    kfn = pl.kernel(
        body,
        out_type=jax.ShapeDtypeStruct((2, NP, d), f32),
        mesh=mesh,
        compiler_params=pltpu.CompilerParams(use_tc_tiling_on_sc=False),
        scratch_types=(
            [pltpu.VMEM_SHARED((NP, d), f32)]
            + [pltpu.VMEM((NBG, CHUNK, d), f32)] * 2
            + [pltpu.VMEM((2, CHUNK, d), f32)]
            + [pltpu.VMEM((2, 16, 128), f32)] * P
            + [pltpu.VMEM((2, QR, CHUNK), jnp.int32)] * 2
            + [pltpu.SemaphoreType.DMA((NBG,)),
               pltpu.SemaphoreType.DMA((NBG,)),
               pltpu.SemaphoreType.DMA((2,)),
               pltpu.SemaphoreType.DMA((2,))]
        ),
    )
    return kfn(A, B, *Es, dstr, srcr)


def kernel(x, edge_index, edge_attr, W1, b1, U1, c1, W2, b2, U2, c2,
           W3, b3, U3, c3):
    dst = edge_index[1].astype(jnp.int32)
    src = edge_index[0].astype(jnp.int32)
    pad = jnp.full((EP - N_EDGES,), N_NODES, jnp.int32)
    dstr = jnp.concatenate([dst, pad]).reshape(ROWS_PAD, CHUNK)
    srcr = jnp.concatenate([src, pad]).reshape(ROWS_PAD, CHUNK)

    x_pad = jnp.zeros((NP, 128), f32).at[:N_NODES].set(x)

    WiT1, WjT1, WeT1 = W1[:, :128].T, W1[:, 128:256].T, W1[:, 256:].T
    WiT2, WjT2, WeT2 = W2[:, :16].T, W2[:, 16:32].T, W2[:, 32:].T
    WiT3, WjT3, WeT3 = W3[:, :32].T, W3[:, 32:64].T, W3[:, 64:].T

    ea_r = edge_attr.reshape(EROWS, 128)
    wbd1 = jnp.kron(jnp.eye(8, dtype=f32), WeT1)
    wbd2 = jnp.kron(jnp.eye(4, dtype=f32), WeT2)
    wbd3 = jnp.kron(jnp.eye(2, dtype=f32), WeT3)
    E1s = [_edge_proj1(ea_r, wbd1, jnp.tile(b1, 8).reshape(1, 128))]
    eouts = _edge_proj23(ea_r, wbd2, jnp.tile(b2, 4).reshape(1, 128),
                         wbd3, jnp.tile(b3, 2).reshape(1, 128))
    E2s, E3s = list(eouts[0:2]), list(eouts[2:6])

    A1, B1, S1 = _node_proj_first(x_pad, WiT1, WjT1, U1.T, c1.reshape(1, -1))
    acc1 = _sc_message_pass(A1, B1, E1s, dstr, srcr, 16)

    A2, B2, S2 = _node_proj_next(acc1, S1, WiT2, WjT2, U2.T, c2.reshape(1, -1))
    acc2 = _sc_message_pass(A2, B2, E2s, dstr, srcr, 32)

    A3, B3, S3 = _node_proj_next(acc2, S2, WiT3, WjT3, U3.T, c3.reshape(1, -1))
    acc3 = _sc_message_pass(A3, B3, E3s, dstr, srcr, 64)

    return _combine(acc3, S3)


# speedup vs baseline: 1.0239x; 1.0239x over previous
import functools

import jax
import jax.numpy as jnp
from jax import lax
from jax.experimental import pallas as pl
from jax.experimental.pallas import tpu as pltpu
from jax.experimental.pallas import tpu_sc as plsc

N_NODES = 10000
N_EDGES = 320000
NP = 10112
STRIPE = NP // 16
CHUNK = 128
ROWS = N_EDGES // CHUNK
NW = 32
ROWS_PAD = 2560
RPW = ROWS_PAD // NW
EP = ROWS_PAD * CHUNK
L = 16

f32 = jnp.float32


EROWS = N_EDGES // 8


def _edge_proj1(ea_r, wbd1, bb1):
    BE = 2000
    blk = pl.BlockSpec((BE, 128), lambda i: (i, 0))
    full = lambda shp: pl.BlockSpec(shp, lambda i: (0, 0))

    def body(ea_ref, w1_ref, b1_ref, o_ref):
        o_ref[...] = (jnp.dot(ea_ref[...], w1_ref[...],
                              preferred_element_type=f32) + b1_ref[...])

    return pl.pallas_call(
        body,
        grid=(EROWS // BE,),
        in_specs=[blk, full((128, 128)), full((1, 128))],
        out_specs=blk,
        out_shape=jax.ShapeDtypeStruct((EROWS, 128), f32),
    )(ea_r, wbd1, bb1)


def _edge_proj23(ea_r, wbd2, bb2, wbd3, bb3):
    BE = 2000
    blk = pl.BlockSpec((BE, 128), lambda i: (i, 0))
    full = lambda shp: pl.BlockSpec(shp, lambda i: (0, 0))

    def body(ea_ref, w2_ref, b2_ref, w3_ref, b3_ref, *outs):
        z = ea_ref[...]
        for x in range(2):
            outs[x][...] = jnp.dot(z[:, 64 * x:64 * x + 64], w2_ref[...],
                                   preferred_element_type=f32) + b2_ref[...]
        for x in range(4):
            outs[2 + x][...] = jnp.dot(z[:, 32 * x:32 * x + 32], w3_ref[...],
                                       preferred_element_type=f32) + b3_ref[...]

    return pl.pallas_call(
        body,
        grid=(EROWS // BE,),
        in_specs=[blk, full((64, 128)), full((1, 128)),
                  full((32, 128)), full((1, 128))],
        out_specs=[blk] * 6,
        out_shape=[jax.ShapeDtypeStruct((EROWS, 128), f32)] * 6,
    )(ea_r, wbd2, bb2, wbd3, bb3)


def _node_proj_first(x, WiT, WjT, UT, cb):
    d = WiT.shape[1]

    def body(x_ref, wi_ref, wj_ref, u_ref, c_ref, a_ref, b_ref, s_ref):
        xb = x_ref[...]
        a_ref[...] = jnp.dot(xb, wi_ref[...], preferred_element_type=f32)
        b_ref[...] = jnp.dot(xb, wj_ref[...], preferred_element_type=f32)
        s_ref[...] = jnp.dot(xb, u_ref[...], preferred_element_type=f32) + c_ref[...]

    return pl.pallas_call(
        body,
        out_shape=[jax.ShapeDtypeStruct((NP, d), f32)] * 3,
    )(x, WiT, WjT, UT, cb)


def _node_proj_next(acc, s_prev, WiT, WjT, UT, cb):
    d = WiT.shape[1]

    def body(acc_ref, sp_ref, wi_ref, wj_ref, u_ref, c_ref, a_ref, b_ref, s_ref):
        xb = acc_ref[0] + acc_ref[1] + sp_ref[...]
        a_ref[...] = jnp.dot(xb, wi_ref[...], preferred_element_type=f32)
        b_ref[...] = jnp.dot(xb, wj_ref[...], preferred_element_type=f32)
        s_ref[...] = jnp.dot(xb, u_ref[...], preferred_element_type=f32) + c_ref[...]

    return pl.pallas_call(
        body,
        out_shape=[jax.ShapeDtypeStruct((NP, d), f32)] * 3,
    )(acc, s_prev, WiT, WjT, UT, cb)


def _combine(acc, s3):
    def body(acc_ref, s_ref, o_ref):
        t = acc_ref[0] + acc_ref[1] + s_ref[...]
        o_ref[...] = t[:N_NODES]

    return pl.pallas_call(
        body,
        out_shape=jax.ShapeDtypeStruct((N_NODES, 64), f32),
    )(acc, s3)


def _sc_message_pass(A, B, Es, dstr, srcr, d):
    KV = d // L
    P = d // 16
    EPR = 8 // P
    mesh = plsc.VectorSubcoreMesh(
        core_axis_name="c", subcore_axis_name="s", num_cores=2, num_subcores=16)

    NBG = 2
    QR = 10
    SPT = d < 64

    def body(*refs):
        (a_hbm, b_hbm), e_hbms = refs[:2], refs[2:2 + P]
        dst_hbm, src_hbm, out_hbm, acc = refs[2 + P:6 + P]
        if SPT:
            atab, btab = refs[6 + P:8 + P]
            rest = refs[8 + P:]
        else:
            atab, btab = a_hbm, b_hbm
            rest = refs[6 + P:]
        abuf, bbuf, mbuf = rest[:3]
        ebufs = rest[3:3 + P]
        idxd, idxs, sa, sb, se, ss = rest[3 + P:]
        cid = lax.axis_index("c")
        sid = lax.axis_index("s")
        wid = cid * 16 + sid
        base_r = wid * RPW

        def zrow(i, carry):
            for k in range(KV):
                mbuf[0, i, pl.ds(k * L, L)] = jnp.zeros((L,), f32)
            return carry
        lax.fori_loop(0, CHUNK, zrow, 0)
        base = sid * STRIPE
        for t in range(STRIPE // CHUNK):
            pltpu.sync_copy(mbuf.at[0], acc.at[pl.ds(base + t * CHUNK, CHUNK)])
        rem = STRIPE % CHUNK
        pltpu.sync_copy(mbuf.at[0, pl.ds(0, rem)],
                        acc.at[pl.ds(base + STRIPE - rem, rem)])

        def load_octet(q):
            qs = q % 2
            pltpu.sync_copy(dst_hbm.at[pl.ds(base_r + q * QR, QR)], idxd.at[qs])
            pltpu.sync_copy(src_hbm.at[pl.ds(base_r + q * QR, QR)], idxs.at[qs])

        def drow(g):
            return idxd.at[(g // QR) % 2, g % QR]

        def srow(g):
            return idxs.at[(g // QR) % 2, g % QR]

        def issue_ab(g):
            gb = g % NBG
            pltpu.async_copy(atab.at[drow(g)], abuf.at[gb], sa.at[gb])
            pltpu.async_copy(btab.at[srow(g)], bbuf.at[gb], sb.at[gb])

        def issue_e(g):
            eb = g % 2

            @pl.when(base_r + g < ROWS)
            def _():
                for x in range(P):
                    pltpu.async_copy(
                        e_hbms[x].at[pl.ds((base_r + g) * 16, 16)],
                        ebufs[x].at[eb], se.at[eb])

        if SPT:
            stt = pl.ds(sid * STRIPE, STRIPE)
            pltpu.sync_copy(a_hbm.at[stt], atab.at[stt])
            pltpu.sync_copy(b_hbm.at[stt], btab.at[stt])
        load_octet(0)
        plsc.subcore_barrier()
        for g0 in range(NBG):
            issue_ab(g0)
        for g0 in range(2):
            issue_e(g0)

        def step(g, carry):
            gb = g % NBG
            mb = g % 2
            pltpu.make_async_copy(atab.at[drow(g)], abuf.at[gb],
                                  sa.at[gb]).wait()
            pltpu.make_async_copy(btab.at[srow(g)], bbuf.at[gb],
                                  sb.at[gb]).wait()

            @pl.when(base_r + g < ROWS)
            def _():
                for x in range(P):
                    pltpu.make_async_copy(
                        e_hbms[x].at[pl.ds((base_r + g) * 16, 16)],
                        ebufs[x].at[mb], se.at[mb]).wait()

            @pl.when(g >= 2)
            def _():
                pltpu.make_async_copy(
                    mbuf.at[mb], acc.at[drow(lax.max(g - 2, 0))],
                    ss.at[mb]).wait()

            def crow(t, c2):
                for x in range(P):
                    for u in range(EPR):
                        e_loc = t * 8 + x * EPR + u
                        for k in range(KV):
                            sl = pl.ds(k * L, L)
                            v = (abuf[gb, e_loc, sl] + bbuf[gb, e_loc, sl]
                                 + ebufs[x][mb, t, pl.ds(u * d + k * L, L)])
                            mbuf[mb, e_loc, sl] = jnp.where(v >= 0.0, v, v * 0.01)
                return c2
            lax.fori_loop(0, 16, crow, 0)

            pltpu.async_copy(mbuf.at[mb], acc.at[drow(g)], ss.at[mb], add=True)

            @pl.when(((g + NBG) % QR == 0) & (g + NBG < RPW))
            def _():
                load_octet((g + NBG) // QR)

            @pl.when(g + NBG < RPW)
            def _():
                issue_ab(g + NBG)

            @pl.when(g + 2 < RPW)
            def _():
                issue_e(g + 2)
            return carry
        lax.fori_loop(0, RPW, step, 0)

        for g in (RPW - 2, RPW - 1):
            pltpu.make_async_copy(mbuf.at[g % 2], acc.at[drow(g)],
                                  ss.at[g % 2]).wait()

        plsc.subcore_barrier()
        pltpu.sync_copy(acc.at[pl.ds(sid * STRIPE, STRIPE)],
                        out_hbm.at[cid, pl.ds(sid * STRIPE, STRIPE)])

    kfn = pl.kernel(
        body,
        out_type=jax.ShapeDtypeStruct((2, NP, d), f32),
        mesh=mesh,
        compiler_params=pltpu.CompilerParams(use_tc_tiling_on_sc=False),
        scratch_types=(
            [pltpu.VMEM_SHARED((NP, d), f32)]
            + ([pltpu.VMEM_SHARED((NP, d), f32)] * 2 if SPT else [])
            + [pltpu.VMEM((NBG, CHUNK, d), f32)] * 2
            + [pltpu.VMEM((2, CHUNK, d), f32)]
            + [pltpu.VMEM((2, 16, 128), f32)] * P
            + [pltpu.VMEM((2, QR, CHUNK), jnp.int32)] * 2
            + [pltpu.SemaphoreType.DMA((NBG,)),
               pltpu.SemaphoreType.DMA((NBG,)),
               pltpu.SemaphoreType.DMA((2,)),
               pltpu.SemaphoreType.DMA((2,))]
        ),
    )
    return kfn(A, B, *Es, dstr, srcr)


def kernel(x, edge_index, edge_attr, W1, b1, U1, c1, W2, b2, U2, c2,
           W3, b3, U3, c3):
    dst = edge_index[1].astype(jnp.int32)
    src = edge_index[0].astype(jnp.int32)
    pad = jnp.full((EP - N_EDGES,), N_NODES, jnp.int32)
    dstr = jnp.concatenate([dst, pad]).reshape(ROWS_PAD, CHUNK)
    srcr = jnp.concatenate([src, pad]).reshape(ROWS_PAD, CHUNK)

    x_pad = jnp.zeros((NP, 128), f32).at[:N_NODES].set(x)

    WiT1, WjT1, WeT1 = W1[:, :128].T, W1[:, 128:256].T, W1[:, 256:].T
    WiT2, WjT2, WeT2 = W2[:, :16].T, W2[:, 16:32].T, W2[:, 32:].T
    WiT3, WjT3, WeT3 = W3[:, :32].T, W3[:, 32:64].T, W3[:, 64:].T

    ea_r = edge_attr.reshape(EROWS, 128)
    wbd1 = jnp.kron(jnp.eye(8, dtype=f32), WeT1)
    wbd2 = jnp.kron(jnp.eye(4, dtype=f32), WeT2)
    wbd3 = jnp.kron(jnp.eye(2, dtype=f32), WeT3)
    E1s = [_edge_proj1(ea_r, wbd1, jnp.tile(b1, 8).reshape(1, 128))]
    eouts = _edge_proj23(ea_r, wbd2, jnp.tile(b2, 4).reshape(1, 128),
                         wbd3, jnp.tile(b3, 2).reshape(1, 128))
    E2s, E3s = list(eouts[0:2]), list(eouts[2:6])

    A1, B1, S1 = _node_proj_first(x_pad, WiT1, WjT1, U1.T, c1.reshape(1, -1))
    acc1 = _sc_message_pass(A1, B1, E1s, dstr, srcr, 16)

    A2, B2, S2 = _node_proj_next(acc1, S1, WiT2, WjT2, U2.T, c2.reshape(1, -1))
    acc2 = _sc_message_pass(A2, B2, E2s, dstr, srcr, 32)

    A3, B3, S3 = _node_proj_next(acc2, S2, WiT3, WjT3, U3.T, c3.reshape(1, -1))
    acc3 = _sc_message_pass(A3, B3, E3s, dstr, srcr, 64)

    return _combine(acc3, S3)


# speedup vs baseline: 1.0409x; 1.0165x over previous
import functools

import jax
import jax.numpy as jnp
from jax import lax
from jax.experimental import pallas as pl
from jax.experimental.pallas import tpu as pltpu
from jax.experimental.pallas import tpu_sc as plsc

N_NODES = 10000
N_EDGES = 320000
NP = 10112
STRIPE = NP // 16
CHUNK = 128
ROWS = N_EDGES // CHUNK
NW = 32
ROWS_PAD = 2560
RPW = ROWS_PAD // NW
EP = ROWS_PAD * CHUNK
L = 16

f32 = jnp.float32


EROWS = N_EDGES // 8


def _edge_proj1(ea_r, wbd1, bb1):
    BE = 2000
    blk = pl.BlockSpec((BE, 128), lambda i: (i, 0))
    full = lambda shp: pl.BlockSpec(shp, lambda i: (0, 0))

    def body(ea_ref, w1_ref, b1_ref, o_ref):
        o_ref[...] = (jnp.dot(ea_ref[...], w1_ref[...],
                              preferred_element_type=f32) + b1_ref[...])

    return pl.pallas_call(
        body,
        grid=(EROWS // BE,),
        in_specs=[blk, full((128, 128)), full((1, 128))],
        out_specs=blk,
        out_shape=jax.ShapeDtypeStruct((EROWS, 128), f32),
    )(ea_r, wbd1, bb1)


def _edge_proj23(ea_r, wbd2, bb2, wbd3, bb3):
    BE = 2000
    blk = pl.BlockSpec((BE, 128), lambda i: (i, 0))
    full = lambda shp: pl.BlockSpec(shp, lambda i: (0, 0))

    def body(ea_ref, w2_ref, b2_ref, w3_ref, b3_ref, *outs):
        z = ea_ref[...]
        for x in range(2):
            outs[x][...] = jnp.dot(z[:, 64 * x:64 * x + 64], w2_ref[...],
                                   preferred_element_type=f32) + b2_ref[...]
        for x in range(4):
            outs[2 + x][...] = jnp.dot(z[:, 32 * x:32 * x + 32], w3_ref[...],
                                       preferred_element_type=f32) + b3_ref[...]

    return pl.pallas_call(
        body,
        grid=(EROWS // BE,),
        in_specs=[blk, full((64, 128)), full((1, 128)),
                  full((32, 128)), full((1, 128))],
        out_specs=[blk] * 6,
        out_shape=[jax.ShapeDtypeStruct((EROWS, 128), f32)] * 6,
    )(ea_r, wbd2, bb2, wbd3, bb3)


def _node_proj_first(x, WiT, WjT, UT, cb):
    d = WiT.shape[1]

    def body(x_ref, wi_ref, wj_ref, u_ref, c_ref, a_ref, b_ref, s_ref):
        xb = x_ref[...]
        a_ref[...] = jnp.dot(xb, wi_ref[...], preferred_element_type=f32)
        b_ref[...] = jnp.dot(xb, wj_ref[...], preferred_element_type=f32)
        s_ref[...] = jnp.dot(xb, u_ref[...], preferred_element_type=f32) + c_ref[...]

    return pl.pallas_call(
        body,
        out_shape=[jax.ShapeDtypeStruct((NP, d), f32)] * 3,
    )(x, WiT, WjT, UT, cb)


def _node_proj_next(acc, s_prev, WiT, WjT, UT, cb):
    d = WiT.shape[1]

    def body(acc_ref, sp_ref, wi_ref, wj_ref, u_ref, c_ref, a_ref, b_ref, s_ref):
        xb = acc_ref[0] + acc_ref[1] + sp_ref[...]
        a_ref[...] = jnp.dot(xb, wi_ref[...], preferred_element_type=f32)
        b_ref[...] = jnp.dot(xb, wj_ref[...], preferred_element_type=f32)
        s_ref[...] = jnp.dot(xb, u_ref[...], preferred_element_type=f32) + c_ref[...]

    return pl.pallas_call(
        body,
        out_shape=[jax.ShapeDtypeStruct((NP, d), f32)] * 3,
    )(acc, s_prev, WiT, WjT, UT, cb)


def _combine(acc, s3):
    def body(acc_ref, s_ref, o_ref):
        t = acc_ref[0] + acc_ref[1] + s_ref[...]
        o_ref[...] = t[:N_NODES]

    return pl.pallas_call(
        body,
        out_shape=jax.ShapeDtypeStruct((N_NODES, 64), f32),
    )(acc, s3)


def _sc_message_pass(A, B, Es, dstr, srcr, d):
    KV = d // L
    P = d // 16
    EPR = 8 // P
    mesh = plsc.VectorSubcoreMesh(
        core_axis_name="c", subcore_axis_name="s", num_cores=2, num_subcores=16)

    def body(*refs):
        (a_hbm, b_hbm), e_hbms = refs[:2], refs[2:2 + P]
        dst_hbm, src_hbm, out_hbm, acc, abuf, bbuf, mbuf = refs[2 + P:9 + P]
        ebufs = refs[9 + P:9 + 2 * P]
        (idxd, idxs, sa0, sa1, sb0, sb1, se0, se1,
         ss0, ss1) = refs[9 + 2 * P:]
        ssems = (ss0, ss1)
        cid = lax.axis_index("c")
        sid = lax.axis_index("s")
        wid = cid * 16 + sid
        base_r = wid * RPW
        sems = ((sa0, sb0, se0), (sa1, sb1, se1))

        def zrow(i, carry):
            for k in range(KV):
                mbuf[0, i, pl.ds(k * L, L)] = jnp.zeros((L,), f32)
            return carry
        lax.fori_loop(0, CHUNK, zrow, 0)
        base = sid * STRIPE
        for t in range(STRIPE // CHUNK):
            pltpu.sync_copy(mbuf.at[0], acc.at[pl.ds(base + t * CHUNK, CHUNK)])
        rem = STRIPE % CHUNK
        pltpu.sync_copy(mbuf.at[0, pl.ds(0, rem)],
                        acc.at[pl.ds(base + STRIPE - rem, rem)])
        pltpu.sync_copy(dst_hbm.at[pl.ds(base_r, RPW)], idxd)
        pltpu.sync_copy(src_hbm.at[pl.ds(base_r, RPW)], idxs)
        plsc.subcore_barrier()

        def issue(g, b):
            sa, sb, se = sems[b]
            pltpu.async_copy(a_hbm.at[idxd.at[g]], abuf.at[b], sa)
            pltpu.async_copy(b_hbm.at[idxs.at[g]], bbuf.at[b], sb)

            @pl.when(base_r + g < ROWS)
            def _():
                for x in range(P):
                    pltpu.async_copy(
                        e_hbms[x].at[pl.ds((base_r + g) * 16, 16)],
                        ebufs[x].at[b], se)

        def wait_chunk(g, b):
            sa, sb, se = sems[b]
            pltpu.make_async_copy(a_hbm.at[idxd.at[g]], abuf.at[b], sa).wait()
            pltpu.make_async_copy(b_hbm.at[idxs.at[g]], bbuf.at[b], sb).wait()

            @pl.when(base_r + g < ROWS)
            def _():
                for x in range(P):
                    pltpu.make_async_copy(
                        e_hbms[x].at[pl.ds((base_r + g) * 16, 16)],
                        ebufs[x].at[b], se).wait()

        issue(0, 0)
        issue(1, 1)

        def outer(go, carry):
            for b in range(2):
                g = go * 2 + b
                wait_chunk(g, b)

                @pl.when(g >= 2)
                def _():
                    pltpu.make_async_copy(
                        mbuf.at[b], acc.at[idxd.at[g - 2]], ssems[b]).wait()

                def crow(t, c2):
                    for x in range(P):
                        for u in range(EPR):
                            e_loc = t * 8 + x * EPR + u
                            for k in range(KV):
                                sl = pl.ds(k * L, L)
                                v = (abuf[b, e_loc, sl] + bbuf[b, e_loc, sl]
                                     + ebufs[x][b, t, pl.ds(u * d + k * L, L)])
                                mbuf[b, e_loc, sl] = jnp.where(v >= 0.0, v, v * 0.01)
                    return c2
                lax.fori_loop(0, 16, crow, 0)

                pltpu.async_copy(mbuf.at[b], acc.at[idxd.at[g]], ssems[b],
                                 add=True)

                @pl.when(g + 2 < RPW)
                def _():
                    issue(g + 2, b)
            return carry
        lax.fori_loop(0, RPW // 2, outer, 0)
        for b in range(2):
            g = RPW - 2 + b
            pltpu.make_async_copy(mbuf.at[b], acc.at[idxd.at[g]],
                                  ssems[b]).wait()

        plsc.subcore_barrier()
        pltpu.sync_copy(acc.at[pl.ds(sid * STRIPE, STRIPE)],
                        out_hbm.at[cid, pl.ds(sid * STRIPE, STRIPE)])

    kfn = pl.kernel(
        body,
        out_type=jax.ShapeDtypeStruct((2, NP, d), f32),
        mesh=mesh,
        compiler_params=pltpu.CompilerParams(use_tc_tiling_on_sc=False),
        scratch_types=(
            [pltpu.VMEM_SHARED((NP, d), f32)]
            + [pltpu.VMEM((2, CHUNK, d), f32)] * 3
            + [pltpu.VMEM((2, 16, 128), f32)] * P
            + [pltpu.VMEM((RPW, CHUNK), jnp.int32)] * 2
            + [pltpu.SemaphoreType.DMA] * 8
        ),
    )
    return kfn(A, B, *Es, dstr, srcr)


def kernel(x, edge_index, edge_attr, W1, b1, U1, c1, W2, b2, U2, c2,
           W3, b3, U3, c3):
    dst = edge_index[1].astype(jnp.int32)
    src = edge_index[0].astype(jnp.int32)
    pad = jnp.full((EP - N_EDGES,), N_NODES, jnp.int32)
    dstr = jnp.concatenate([dst, pad]).reshape(ROWS_PAD, CHUNK)
    srcr = jnp.concatenate([src, pad]).reshape(ROWS_PAD, CHUNK)

    x_pad = jnp.zeros((NP, 128), f32).at[:N_NODES].set(x)

    WiT1, WjT1, WeT1 = W1[:, :128].T, W1[:, 128:256].T, W1[:, 256:].T
    WiT2, WjT2, WeT2 = W2[:, :16].T, W2[:, 16:32].T, W2[:, 32:].T
    WiT3, WjT3, WeT3 = W3[:, :32].T, W3[:, 32:64].T, W3[:, 64:].T

    ea_r = edge_attr.reshape(EROWS, 128)
    wbd1 = jnp.kron(jnp.eye(8, dtype=f32), WeT1)
    wbd2 = jnp.kron(jnp.eye(4, dtype=f32), WeT2)
    wbd3 = jnp.kron(jnp.eye(2, dtype=f32), WeT3)
    E1s = [_edge_proj1(ea_r, wbd1, jnp.tile(b1, 8).reshape(1, 128))]
    eouts = _edge_proj23(ea_r, wbd2, jnp.tile(b2, 4).reshape(1, 128),
                         wbd3, jnp.tile(b3, 2).reshape(1, 128))
    E2s, E3s = list(eouts[0:2]), list(eouts[2:6])

    A1, B1, S1 = _node_proj_first(x_pad, WiT1, WjT1, U1.T, c1.reshape(1, -1))
    acc1 = _sc_message_pass(A1, B1, E1s, dstr, srcr, 16)

    A2, B2, S2 = _node_proj_next(acc1, S1, WiT2, WjT2, U2.T, c2.reshape(1, -1))
    acc2 = _sc_message_pass(A2, B2, E2s, dstr, srcr, 32)

    A3, B3, S3 = _node_proj_next(acc2, S2, WiT3, WjT3, U3.T, c3.reshape(1, -1))
    acc3 = _sc_message_pass(A3, B3, E3s, dstr, srcr, 64)

    return _combine(acc3, S3)
